# baseline (speedup 1.0000x reference)
import functools

import jax
import jax.numpy as jnp
from jax import lax
from jax.experimental import pallas as pl
from jax.experimental.pallas import tpu as pltpu
from jax.experimental.pallas import tpu_sc as plsc

_B = 1024
_EMB = 128
_K = 10
_RT = 256
_NW = 32
_RPW = _B // _NW
_BATCH = 8
_LANES = 16


def _dist_body(t_ref, a_ref, v_ref, tt_ref, at_ref, vt_ref, dist_ref,
               h_scr, ht_scr):
    step = pl.program_id(0)

    C = jnp.float32(0.333984375)

    def _h(x, y, z):
        xb = x.astype(jnp.bfloat16).astype(jnp.float32)
        yb = y.astype(jnp.bfloat16).astype(jnp.float32)
        zb = z.astype(jnp.bfloat16).astype(jnp.float32)
        return (C * xb + C * yb) + C * zb

    @pl.when(step == 0)
    def _():
        h_scr[...] = _h(t_ref[...], a_ref[...], v_ref[...])
        ht_scr[...] = _h(tt_ref[...], at_ref[...], vt_ref[...])

    x = h_scr[pl.ds(step * _RT, _RT), :]

    UK = 8
    sub = lax.broadcasted_iota(jnp.int32, (UK, _EMB), 0)
    lane = lax.broadcasted_iota(jnp.int32, (UK, _EMB), 1)

    def kbody(kk, acc):
        yblk = ht_scr[pl.ds(kk * UK, UK), :]
        oh = jnp.where(lane == kk * UK + sub, 1.0, 0.0)
        xblk = lax.dot_general(x, oh, (((1,), (1,)), ((), ())),
                               precision=lax.Precision.HIGHEST,
                               preferred_element_type=jnp.float32)
        part = jnp.zeros((_RT, _B), jnp.float32)
        for u in range(UK):
            xk = lax.slice(xblk, (0, u), (_RT, u + 1))
            yk = lax.slice(yblk, (u, 0), (u + 1, _B))
            part = part + jnp.abs(xk - yk)
        return acc + part

    dist_ref[...] = lax.fori_loop(
        0, _EMB // UK, kbody, jnp.zeros((_RT, _B), jnp.float32))


def _sc_topk_scatter_body(dist_hbm, ct_hbm, din, cout):
    core = lax.axis_index("c")
    sub = lax.axis_index("s")
    wid = sub * 2 + core
    base = wid * _RPW
    lanes = lax.iota(jnp.int32, _LANES)
    ones16 = jnp.ones((_LANES,), jnp.float32)

    for b in range(_RPW // _BATCH):
        row0 = base + b * _BATCH
        pltpu.sync_copy(dist_hbm.at[pl.ds(row0, _BATCH)], din)
        for i in range(_BATCH):
            def zero_body(c, _):
                cout[i, pl.ds(c * _LANES, _LANES)] = (
                    jnp.zeros((_LANES,), jnp.float32))
                return 0
            lax.fori_loop(0, _B // _LANES, zero_body, 0)

            v0 = din[i, pl.ds(0, _LANES)]
            cur, curi = plsc.sort_key_val(v0, lanes, descending=True)

            def chunk_body(c, carry):
                cur, curi, thr = carry
                vals = din[i, pl.ds(c * _LANES, _LANES)]
                vidx = c * _LANES + lanes
                anyg = jnp.any(vals > thr)

                def merge(ops):
                    cur, curi, vals, vidx = ops
                    sv, si = plsc.sort_key_val(vals, vidx, descending=True)
                    rk = lax.rev(sv, (0,))
                    ri = lax.rev(si, (0,))
                    mv = jnp.maximum(cur, rk)
                    mi = jnp.where(cur >= rk, curi, ri)
                    nk, nv = plsc.sort_key_val(mv, mi, descending=True)
                    return (nk, nv)

                def keep(ops):
                    return (ops[0], ops[1])

                cur, curi = lax.cond(anyg, merge, keep,
                                     (cur, curi, vals, vidx))
                return (cur, curi, jnp.min(cur))

            cur, curi, _thr = lax.fori_loop(
                1, _B // _LANES, chunk_body, (cur, curi, jnp.min(cur)))

            r = row0 + i
            idxv = jnp.where(lanes < _K, curi, r)
            msk = lanes <= _K
            rowv = jnp.full((_LANES,), i, jnp.int32)
            plsc.store_scatter(cout, [rowv, idxv], ones16, mask=msk)
        pltpu.sync_copy(cout, ct_hbm.at[pl.ds(row0, _BATCH)])


def _adjacency_body(ct_ref, out_ref, a_scr):
    i = pl.program_id(0)
    j = pl.program_id(1)

    @pl.when((i == 0) & (j == 0))
    def _():
        ct = ct_ref[...]
        s = jnp.sum(ct, axis=1, keepdims=True)
        w = ct / s
        a0 = lax.dot_general(w, ct, (((0,), (0,)), ((), ())),
                             precision=lax.Precision.HIGHEST,
                             preferred_element_type=jnp.float32)
        ones = jnp.ones((_B, 1), jnp.float32)
        dcol = lax.dot_general(ct, ones, (((0,), (0,)), ((), ())),
                               precision=lax.Precision.HIGHEST)
        ii = lax.broadcasted_iota(jnp.int32, (_B, _B), 0)
        jj = lax.broadcasted_iota(jnp.int32, (_B, _B), 1)
        eye = jnp.where(ii == jj, 1.0, 0.0)
        a_scr[...] = (a0 + eye) / (3.0 * (1.0 + dcol))

    out_ref[...] = a_scr[...]


@functools.cache
def _build_calls():
    full_spec = lambda shape: pl.BlockSpec(shape, lambda s: (0, 0))
    dist = pl.pallas_call(
        _dist_body,
        grid=(_B // _RT,),
        in_specs=[
            full_spec((_B, _EMB)), full_spec((_B, _EMB)), full_spec((_B, _EMB)),
            full_spec((_EMB, _B)), full_spec((_EMB, _B)), full_spec((_EMB, _B)),
        ],
        out_specs=pl.BlockSpec((_RT, _B), lambda s: (s, 0)),
        out_shape=jax.ShapeDtypeStruct((_B, _B), jnp.float32),
        scratch_shapes=[
            pltpu.VMEM((_B, _EMB), jnp.float32),
            pltpu.VMEM((_EMB, _B), jnp.float32),
        ],
    )
    mesh = plsc.VectorSubcoreMesh(core_axis_name="c", subcore_axis_name="s")
    sc_topk_scatter = functools.partial(
        pl.kernel,
        out_type=jax.ShapeDtypeStruct((_B, _B), jnp.float32),
        mesh=mesh,
        scratch_types=[
            pltpu.VMEM((_BATCH, _B), jnp.float32),
            pltpu.VMEM((_BATCH, _B), jnp.float32),
        ],
        compiler_params=pltpu.CompilerParams(needs_layout_passes=False),
    )(_sc_topk_scatter_body)
    adjacency = pl.pallas_call(
        _adjacency_body,
        grid=(3, 3),
        in_specs=[pl.BlockSpec((_B, _B), lambda i, j: (0, 0))],
        out_specs=pl.BlockSpec((_B, _B), lambda i, j: (i, j)),
        out_shape=jax.ShapeDtypeStruct((3 * _B, 3 * _B), jnp.float32),
        scratch_shapes=[
            pltpu.VMEM((_B, _B), jnp.float32),
        ],
    )
    return dist, sc_topk_scatter, adjacency


@jax.jit
def kernel(nodes_t, nodes_a, nodes_v, batch_size):
    del batch_size
    t = nodes_t.astype(jnp.float32)
    a = nodes_a.astype(jnp.float32)
    v = nodes_v.astype(jnp.float32)
    dist_fn, sc_fn, adj_fn = _build_calls()
    dist = dist_fn(t, a, v, t.T, a.T, v.T)
    ct = sc_fn(dist)
    adjacency = adj_fn(ct)
    nodes_list = jnp.concatenate([t, a, v], axis=0)
    return adjacency, nodes_list


# speedup vs baseline: 1.0323x; 1.0323x over previous
import functools

import jax
import jax.numpy as jnp
from jax import lax
from jax.experimental import pallas as pl
from jax.experimental.pallas import tpu as pltpu
from jax.experimental.pallas import tpu_sc as plsc

_B = 1024
_EMB = 128
_K = 10
_RT = 256
_NW = 32
_RPW = _B // _NW
_BATCH = 8
_LANES = 16


def _dist_body(t_ref, a_ref, v_ref, tt_ref, at_ref, vt_ref, dist_ref,
               h_scr, ht_scr):
    step = pl.program_id(0)

    C = jnp.float32(0.333984375)

    def _h(x, y, z):
        xb = x.astype(jnp.bfloat16).astype(jnp.float32)
        yb = y.astype(jnp.bfloat16).astype(jnp.float32)
        zb = z.astype(jnp.bfloat16).astype(jnp.float32)
        return (C * xb + C * yb) + C * zb

    @pl.when(step == 0)
    def _():
        h_scr[...] = _h(t_ref[...], a_ref[...], v_ref[...])
        ht_scr[...] = _h(tt_ref[...], at_ref[...], vt_ref[...])

    x = h_scr[pl.ds(step * _RT, _RT), :]

    UK = 8
    sub = lax.broadcasted_iota(jnp.int32, (UK, _EMB), 0)
    lane = lax.broadcasted_iota(jnp.int32, (UK, _EMB), 1)

    def kbody(kk, acc):
        yblk = ht_scr[pl.ds(kk * UK, UK), :]
        oh = jnp.where(lane == kk * UK + sub, 1.0, 0.0)
        xblk = lax.dot_general(x, oh, (((1,), (1,)), ((), ())),
                               precision=lax.Precision.HIGHEST,
                               preferred_element_type=jnp.float32)
        part = jnp.zeros((_RT, _B), jnp.float32)
        for u in range(UK):
            xk = lax.slice(xblk, (0, u), (_RT, u + 1))
            yk = lax.slice(yblk, (u, 0), (u + 1, _B))
            part = part + jnp.abs(xk - yk)
        return acc + part

    dist_ref[...] = lax.fori_loop(
        0, _EMB // UK, kbody, jnp.zeros((_RT, _B), jnp.float32))


def _sc_topk_scatter_body(dist_hbm, ct_hbm, din, cout):
    core = lax.axis_index("c")
    sub = lax.axis_index("s")
    wid = sub * 2 + core
    base = wid * _RPW
    lanes = lax.iota(jnp.int32, _LANES)
    ones16 = jnp.ones((_LANES,), jnp.float32)

    for b in range(_RPW // _BATCH):
        row0 = base + b * _BATCH
        pltpu.sync_copy(dist_hbm.at[pl.ds(row0, _BATCH)], din)
        for i in range(_BATCH):
            def zero_body(c, _):
                cout[i, pl.ds(c * _LANES, _LANES)] = (
                    jnp.zeros((_LANES,), jnp.float32))
                return 0
            lax.fori_loop(0, _B // _LANES, zero_body, 0)

            GRP = 4

            def merge(ops):
                cur, curi, _thr, vals, vidx = ops
                sv, si = plsc.sort_key_val(vals, vidx, descending=True)
                rk = lax.rev(sv, (0,))
                ri = lax.rev(si, (0,))
                mv = jnp.maximum(cur, rk)
                mi = jnp.where(cur >= rk, curi, ri)
                nk, nv = plsc.sort_key_val(mv, mi, descending=True)
                return (nk, nv, jnp.min(nk))

            def keep(ops):
                return (ops[0], ops[1], ops[2])

            def group_body(g, carry):
                cur, curi, thr = carry
                vs = [din[i, pl.ds(g * (GRP * _LANES) + u * _LANES, _LANES)]
                      for u in range(GRP)]
                gm = jnp.maximum(jnp.maximum(vs[0], vs[1]),
                                 jnp.maximum(vs[2], vs[3]))
                anyg = jnp.any(gm > thr)

                def do_group(ops):
                    cur, curi, thr = ops
                    for u in range(GRP):
                        vidx = g * (GRP * _LANES) + u * _LANES + lanes
                        anyc = jnp.any(vs[u] > thr)
                        cur, curi, thr = lax.cond(
                            anyc, merge, keep, (cur, curi, thr, vs[u], vidx))
                    return (cur, curi, thr)

                return lax.cond(anyg, do_group, keep, (cur, curi, thr))

            neg = jnp.full((_LANES,), -1.0, jnp.float32)
            cur, curi, _thr = lax.fori_loop(
                0, _B // (GRP * _LANES), group_body,
                (neg, jnp.zeros((_LANES,), jnp.int32), jnp.float32(-1.0)))

            r = row0 + i
            idxv = jnp.where(lanes < _K, curi, r)
            msk = lanes <= _K
            rowv = jnp.full((_LANES,), i, jnp.int32)
            plsc.store_scatter(cout, [rowv, idxv], ones16, mask=msk)
        pltpu.sync_copy(cout, ct_hbm.at[pl.ds(row0, _BATCH)])


def _adjacency_body(ct_ref, out_ref, a_scr):
    i = pl.program_id(0)
    j = pl.program_id(1)

    @pl.when((i == 0) & (j == 0))
    def _():
        ct = ct_ref[...]
        s = jnp.sum(ct, axis=1, keepdims=True)
        w = ct / s
        a0 = lax.dot_general(w, ct, (((0,), (0,)), ((), ())),
                             precision=lax.Precision.HIGHEST,
                             preferred_element_type=jnp.float32)
        ones = jnp.ones((_B, 1), jnp.float32)
        dcol = lax.dot_general(ct, ones, (((0,), (0,)), ((), ())),
                               precision=lax.Precision.HIGHEST)
        ii = lax.broadcasted_iota(jnp.int32, (_B, _B), 0)
        jj = lax.broadcasted_iota(jnp.int32, (_B, _B), 1)
        eye = jnp.where(ii == jj, 1.0, 0.0)
        a_scr[...] = (a0 + eye) / (3.0 * (1.0 + dcol))

    out_ref[...] = a_scr[...]


@functools.cache
def _build_calls():
    full_spec = lambda shape: pl.BlockSpec(shape, lambda s: (0, 0))
    dist = pl.pallas_call(
        _dist_body,
        grid=(_B // _RT,),
        in_specs=[
            full_spec((_B, _EMB)), full_spec((_B, _EMB)), full_spec((_B, _EMB)),
            full_spec((_EMB, _B)), full_spec((_EMB, _B)), full_spec((_EMB, _B)),
        ],
        out_specs=pl.BlockSpec((_RT, _B), lambda s: (s, 0)),
        out_shape=jax.ShapeDtypeStruct((_B, _B), jnp.float32),
        scratch_shapes=[
            pltpu.VMEM((_B, _EMB), jnp.float32),
            pltpu.VMEM((_EMB, _B), jnp.float32),
        ],
    )
    mesh = plsc.VectorSubcoreMesh(core_axis_name="c", subcore_axis_name="s")
    sc_topk_scatter = functools.partial(
        pl.kernel,
        out_type=jax.ShapeDtypeStruct((_B, _B), jnp.float32),
        mesh=mesh,
        scratch_types=[
            pltpu.VMEM((_BATCH, _B), jnp.float32),
            pltpu.VMEM((_BATCH, _B), jnp.float32),
        ],
        compiler_params=pltpu.CompilerParams(needs_layout_passes=False),
    )(_sc_topk_scatter_body)
    adjacency = pl.pallas_call(
        _adjacency_body,
        grid=(3, 3),
        in_specs=[pl.BlockSpec((_B, _B), lambda i, j: (0, 0))],
        out_specs=pl.BlockSpec((_B, _B), lambda i, j: (i, j)),
        out_shape=jax.ShapeDtypeStruct((3 * _B, 3 * _B), jnp.float32),
        scratch_shapes=[
            pltpu.VMEM((_B, _B), jnp.float32),
        ],
    )
    return dist, sc_topk_scatter, adjacency


@jax.jit
def kernel(nodes_t, nodes_a, nodes_v, batch_size):
    del batch_size
    t = nodes_t.astype(jnp.float32)
    a = nodes_a.astype(jnp.float32)
    v = nodes_v.astype(jnp.float32)
    dist_fn, sc_fn, adj_fn = _build_calls()
    dist = dist_fn(t, a, v, t.T, a.T, v.T)
    ct = sc_fn(dist)
    adjacency = adj_fn(ct)
    nodes_list = jnp.concatenate([t, a, v], axis=0)
    return adjacency, nodes_list


# speedup vs baseline: 1.4531x; 1.4075x over previous
import functools

import jax
import jax.numpy as jnp
from jax import lax
from jax.experimental import pallas as pl
from jax.experimental.pallas import tpu as pltpu
from jax.experimental.pallas import tpu_sc as plsc

_B = 1024
_EMB = 128
_K = 10
_RT = 256
_NW = 32
_RPW = _B // _NW
_BATCH = 8
_LANES = 16
_NCHUNK = _B // _LANES
_TOPC = 11


def _dist_body(t_ref, a_ref, v_ref, tt_ref, at_ref, vt_ref, dist_ref, cm_ref,
               h_scr, ht_scr):
    step = pl.program_id(0)

    C = jnp.float32(0.333984375)

    def _h(x, y, z):
        xb = x.astype(jnp.bfloat16).astype(jnp.float32)
        yb = y.astype(jnp.bfloat16).astype(jnp.float32)
        zb = z.astype(jnp.bfloat16).astype(jnp.float32)
        return (C * xb + C * yb) + C * zb

    @pl.when(step == 0)
    def _():
        h_scr[...] = _h(t_ref[...], a_ref[...], v_ref[...])
        ht_scr[...] = _h(tt_ref[...], at_ref[...], vt_ref[...])

    x = h_scr[pl.ds(step * _RT, _RT), :]

    UK = 8
    sub = lax.broadcasted_iota(jnp.int32, (UK, _EMB), 0)
    lane = lax.broadcasted_iota(jnp.int32, (UK, _EMB), 1)

    def kbody(kk, acc):
        yblk = ht_scr[pl.ds(kk * UK, UK), :]
        oh = jnp.where(lane == kk * UK + sub, 1.0, 0.0)
        xblk = lax.dot_general(x, oh, (((1,), (1,)), ((), ())),
                               precision=lax.Precision.HIGHEST,
                               preferred_element_type=jnp.float32)
        part = jnp.zeros((_RT, _B), jnp.float32)
        for u in range(UK):
            xk = lax.slice(xblk, (0, u), (_RT, u + 1))
            yk = lax.slice(yblk, (u, 0), (u + 1, _B))
            part = part + jnp.abs(xk - yk)
        return acc + part

    acc = lax.fori_loop(0, _EMB // UK, kbody,
                        jnp.zeros((_RT, _B), jnp.float32))
    dist_ref[...] = acc
    cm_ref[...] = jnp.max(acc.reshape(_RT // _LANES, _LANES, _B), axis=1)


def _sc_topk_scatter_body(dist_hbm, cm_hbm, ct_hbm, din, cout, cmv, ixb):
    core = lax.axis_index("c")
    sub = lax.axis_index("s")
    wid = sub * 2 + core
    base = wid * _RPW
    lanes = lax.iota(jnp.int32, _LANES)
    ones16 = jnp.ones((_LANES,), jnp.float32)
    zeros16 = jnp.zeros((_LANES,), jnp.float32)

    pltpu.sync_copy(cm_hbm, cmv)

    def zbody(c, _):
        cout[pl.ds(c * _LANES, _LANES)] = zeros16
        return 0
    lax.fori_loop(0, _BATCH * _B // _LANES, zbody, 0)

    def bmerge(a, ai, b, bi):
        rb = lax.rev(b, (0,))
        rbi = lax.rev(bi, (0,))
        mv = jnp.maximum(a, rb)
        mi = jnp.where(a >= rb, ai, rbi)
        nk, nv = plsc.sort_key_val(mv, mi, descending=True)
        return nk, nv

    for b in range(_RPW // _BATCH):
        row0 = base + b * _BATCH
        pltpu.sync_copy(dist_hbm.at[pl.ds(row0 * _B, _BATCH * _B)], din)

        def row_body(i, _):
            r = row0 + i
            ms, ids = [], []
            for k in range(_NCHUNK // _LANES):
                cid = k * _LANES + lanes
                mk = plsc.load_gather(cmv, [cid * _B + r])
                sk, sid = plsc.sort_key_val(mk, cid, descending=True)
                ms.append(sk)
                ids.append(sid)
            t0, t0i = bmerge(ms[0], ids[0], ms[1], ids[1])
            t1, t1i = bmerge(ms[2], ids[2], ms[3], ids[3])
            _sm, smi = bmerge(t0, t0i, t1, t1i)

            def merge_step(k, carry):
                cur, curi = carry
                ck = jnp.max(jnp.where(lanes == k, smi, -1))
                vidx = ck * _LANES + lanes
                vals = plsc.load_gather(din, [i * _B + vidx])
                sv, si = plsc.sort_key_val(vals, vidx, descending=True)
                return bmerge(cur, curi, sv, si)

            cur, curi = lax.fori_loop(
                0, _TOPC, merge_step,
                (jnp.full((_LANES,), -1.0, jnp.float32),
                 jnp.zeros((_LANES,), jnp.int32)))

            cols = jnp.where(lanes < _K, curi, r)
            idxv = i * _B + cols
            plsc.store_scatter(cout, [idxv], ones16, mask=lanes <= _K)
            ixb[pl.ds(i * _LANES, _LANES)] = idxv
            return 0

        lax.fori_loop(0, _BATCH, row_body, 0)
        pltpu.sync_copy(cout, ct_hbm.at[pl.ds(row0 * _B, _BATCH * _B)])

        def unscatter(i, _):
            iv = ixb[pl.ds(i * _LANES, _LANES)]
            plsc.store_scatter(cout, [iv], zeros16, mask=lanes <= _K)
            return 0
        lax.fori_loop(0, _BATCH, unscatter, 0)


def _adjacency_body(ct_ref, out_ref, a_scr):
    i = pl.program_id(0)
    j = pl.program_id(1)

    @pl.when((i == 0) & (j == 0))
    def _():
        ct = ct_ref[...]
        s = jnp.sum(ct, axis=1, keepdims=True)
        w = ct / s
        a0 = lax.dot_general(w, ct, (((0,), (0,)), ((), ())),
                             precision=lax.Precision.HIGHEST,
                             preferred_element_type=jnp.float32)
        ones = jnp.ones((_B, 1), jnp.float32)
        dcol = lax.dot_general(ct, ones, (((0,), (0,)), ((), ())),
                               precision=lax.Precision.HIGHEST)
        ii = lax.broadcasted_iota(jnp.int32, (_B, _B), 0)
        jj = lax.broadcasted_iota(jnp.int32, (_B, _B), 1)
        eye = jnp.where(ii == jj, 1.0, 0.0)
        a_scr[...] = (a0 + eye) / (3.0 * (1.0 + dcol))

    out_ref[...] = a_scr[...]


@functools.cache
def _build_calls():
    full_spec = lambda shape: pl.BlockSpec(shape, lambda s: (0, 0))
    dist = pl.pallas_call(
        _dist_body,
        grid=(_B // _RT,),
        in_specs=[
            full_spec((_B, _EMB)), full_spec((_B, _EMB)), full_spec((_B, _EMB)),
            full_spec((_EMB, _B)), full_spec((_EMB, _B)), full_spec((_EMB, _B)),
        ],
        out_specs=[
            pl.BlockSpec((_RT, _B), lambda s: (s, 0)),
            pl.BlockSpec((_RT // _LANES, _B), lambda s: (s, 0)),
        ],
        out_shape=[
            jax.ShapeDtypeStruct((_B, _B), jnp.float32),
            jax.ShapeDtypeStruct((_NCHUNK, _B), jnp.float32),
        ],
        scratch_shapes=[
            pltpu.VMEM((_B, _EMB), jnp.float32),
            pltpu.VMEM((_EMB, _B), jnp.float32),
        ],
    )
    mesh = plsc.VectorSubcoreMesh(core_axis_name="c", subcore_axis_name="s")
    sc_topk_scatter = functools.partial(
        pl.kernel,
        out_type=jax.ShapeDtypeStruct((_B * _B,), jnp.float32),
        mesh=mesh,
        scratch_types=[
            pltpu.VMEM((_BATCH * _B,), jnp.float32),
            pltpu.VMEM((_BATCH * _B,), jnp.float32),
            pltpu.VMEM((_NCHUNK * _B,), jnp.float32),
            pltpu.VMEM((_BATCH * _LANES,), jnp.int32),
        ],
        compiler_params=pltpu.CompilerParams(needs_layout_passes=False),
    )(_sc_topk_scatter_body)
    adjacency = pl.pallas_call(
        _adjacency_body,
        grid=(3, 3),
        in_specs=[pl.BlockSpec((_B, _B), lambda i, j: (0, 0))],
        out_specs=pl.BlockSpec((_B, _B), lambda i, j: (i, j)),
        out_shape=jax.ShapeDtypeStruct((3 * _B, 3 * _B), jnp.float32),
        scratch_shapes=[
            pltpu.VMEM((_B, _B), jnp.float32),
        ],
    )
    return dist, sc_topk_scatter, adjacency


@jax.jit
def kernel(nodes_t, nodes_a, nodes_v, batch_size):
    del batch_size
    t = nodes_t.astype(jnp.float32)
    a = nodes_a.astype(jnp.float32)
    v = nodes_v.astype(jnp.float32)
    dist_fn, sc_fn, adj_fn = _build_calls()
    dist, cm = dist_fn(t, a, v, t.T, a.T, v.T)
    ct_flat = sc_fn(dist.reshape(-1), cm.reshape(-1))
    adjacency = adj_fn(ct_flat.reshape(_B, _B))
    nodes_list = jnp.concatenate([t, a, v], axis=0)
    return adjacency, nodes_list


# speedup vs baseline: 1.5327x; 1.0548x over previous
import functools

import jax
import jax.numpy as jnp
from jax import lax
from jax.experimental import pallas as pl
from jax.experimental.pallas import tpu as pltpu
from jax.experimental.pallas import tpu_sc as plsc

_B = 1024
_EMB = 128
_K = 10
_RT = 256
_NW = 32
_RPW = _B // _NW
_BATCH = 8
_LANES = 16
_NCHUNK = _B // _LANES
_TOPC = 11


def _dist_body(t_ref, a_ref, v_ref, tt_ref, at_ref, vt_ref, dist_ref, cm_ref,
               h_scr, ht_scr):
    step = pl.program_id(0)

    C = jnp.float32(0.333984375)

    def _h(x, y, z):
        xb = x.astype(jnp.bfloat16).astype(jnp.float32)
        yb = y.astype(jnp.bfloat16).astype(jnp.float32)
        zb = z.astype(jnp.bfloat16).astype(jnp.float32)
        return (C * xb + C * yb) + C * zb

    @pl.when(step == 0)
    def _():
        h_scr[...] = _h(t_ref[...], a_ref[...], v_ref[...])
        ht_scr[...] = _h(tt_ref[...], at_ref[...], vt_ref[...])

    x = h_scr[pl.ds(step * _RT, _RT), :]

    UK = 8
    sub = lax.broadcasted_iota(jnp.int32, (UK, _EMB), 0)
    lane = lax.broadcasted_iota(jnp.int32, (UK, _EMB), 1)

    def kbody(kk, acc):
        yblk = ht_scr[pl.ds(kk * UK, UK), :]
        oh = jnp.where(lane == kk * UK + sub, 1.0, 0.0)
        xblk = lax.dot_general(x, oh, (((1,), (1,)), ((), ())),
                               precision=lax.Precision.HIGHEST,
                               preferred_element_type=jnp.float32)
        part = jnp.zeros((_RT, _B), jnp.float32)
        for u in range(UK):
            xk = lax.slice(xblk, (0, u), (_RT, u + 1))
            yk = lax.slice(yblk, (u, 0), (u + 1, _B))
            part = part + jnp.abs(xk - yk)
        return acc + part

    acc = lax.fori_loop(0, _EMB // UK, kbody,
                        jnp.zeros((_RT, _B), jnp.float32))
    dist_ref[...] = acc
    cm_ref[...] = jnp.max(acc.reshape(_RT // _LANES, _LANES, _B), axis=1)


def _sc_topk_scatter_body(dist_hbm, cm_hbm, ct_hbm, din, cout, cmv, ixb):
    core = lax.axis_index("c")
    sub = lax.axis_index("s")
    wid = sub * 2 + core
    base = wid * _RPW
    lanes = lax.iota(jnp.int32, _LANES)
    ones16 = jnp.ones((_LANES,), jnp.float32)
    zeros16 = jnp.zeros((_LANES,), jnp.float32)

    pltpu.sync_copy(cm_hbm.at[pl.ds(base * _NCHUNK, _RPW * _NCHUNK)], cmv)

    def zbody(c, _):
        cout[pl.ds(c * _LANES, _LANES)] = zeros16
        return 0
    lax.fori_loop(0, _BATCH * _B // _LANES, zbody, 0)

    def bmerge(a, ai, b, bi):
        rb = lax.rev(b, (0,))
        rbi = lax.rev(bi, (0,))
        mv = jnp.maximum(a, rb)
        mi = jnp.where(a >= rb, ai, rbi)
        nk, nv = plsc.sort_key_val(mv, mi, descending=True)
        return nk, nv

    for b in range(_RPW // _BATCH):
        row0 = base + b * _BATCH
        pltpu.sync_copy(dist_hbm.at[pl.ds(row0 * _B, _BATCH * _B)], din)

        def row_body(i, _):
            r = row0 + i
            ms, ids = [], []
            li = b * _BATCH + i
            for k in range(_NCHUNK // _LANES):
                cid = k * _LANES + lanes
                mk = cmv[pl.ds(li * _NCHUNK + k * _LANES, _LANES)]
                sk, sid = plsc.sort_key_val(mk, cid, descending=True)
                ms.append(sk)
                ids.append(sid)
            t0, t0i = bmerge(ms[0], ids[0], ms[1], ids[1])
            t1, t1i = bmerge(ms[2], ids[2], ms[3], ids[3])
            _sm, smi = bmerge(t0, t0i, t1, t1i)

            def merge_step(k, carry):
                cur, curi = carry
                ck = jnp.max(jnp.where(lanes == k, smi, -1))
                vidx = ck * _LANES + lanes
                vals = plsc.load_gather(din, [i * _B + vidx])
                sv, si = plsc.sort_key_val(vals, vidx, descending=True)
                return bmerge(cur, curi, sv, si)

            cur, curi = lax.fori_loop(
                0, _TOPC, merge_step,
                (jnp.full((_LANES,), -1.0, jnp.float32),
                 jnp.zeros((_LANES,), jnp.int32)))

            cols = jnp.where(lanes < _K, curi, r)
            idxv = i * _B + cols
            plsc.store_scatter(cout, [idxv], ones16, mask=lanes <= _K)
            ixb[pl.ds(i * _LANES, _LANES)] = idxv
            return 0

        lax.fori_loop(0, _BATCH, row_body, 0)
        pltpu.sync_copy(cout, ct_hbm.at[pl.ds(row0 * _B, _BATCH * _B)])

        def unscatter(i, _):
            iv = ixb[pl.ds(i * _LANES, _LANES)]
            plsc.store_scatter(cout, [iv], zeros16, mask=lanes <= _K)
            return 0
        lax.fori_loop(0, _BATCH, unscatter, 0)


def _adjacency_body(ct_ref, out_ref, a_scr):
    i = pl.program_id(0)
    j = pl.program_id(1)

    @pl.when((i == 0) & (j == 0))
    def _():
        ct = ct_ref[...]
        s = jnp.sum(ct, axis=1, keepdims=True)
        w = ct / s
        a0 = lax.dot_general(w, ct, (((0,), (0,)), ((), ())),
                             precision=lax.Precision.HIGHEST,
                             preferred_element_type=jnp.float32)
        ones = jnp.ones((_B, 1), jnp.float32)
        dcol = lax.dot_general(ct, ones, (((0,), (0,)), ((), ())),
                               precision=lax.Precision.HIGHEST)
        ii = lax.broadcasted_iota(jnp.int32, (_B, _B), 0)
        jj = lax.broadcasted_iota(jnp.int32, (_B, _B), 1)
        eye = jnp.where(ii == jj, 1.0, 0.0)
        a_scr[...] = (a0 + eye) / (3.0 * (1.0 + dcol))

    out_ref[...] = a_scr[...]


@functools.cache
def _build_calls():
    full_spec = lambda shape: pl.BlockSpec(shape, lambda s: (0, 0))
    dist = pl.pallas_call(
        _dist_body,
        grid=(_B // _RT,),
        in_specs=[
            full_spec((_B, _EMB)), full_spec((_B, _EMB)), full_spec((_B, _EMB)),
            full_spec((_EMB, _B)), full_spec((_EMB, _B)), full_spec((_EMB, _B)),
        ],
        out_specs=[
            pl.BlockSpec((_RT, _B), lambda s: (s, 0)),
            pl.BlockSpec((_RT // _LANES, _B), lambda s: (s, 0)),
        ],
        out_shape=[
            jax.ShapeDtypeStruct((_B, _B), jnp.float32),
            jax.ShapeDtypeStruct((_NCHUNK, _B), jnp.float32),
        ],
        scratch_shapes=[
            pltpu.VMEM((_B, _EMB), jnp.float32),
            pltpu.VMEM((_EMB, _B), jnp.float32),
        ],
    )
    mesh = plsc.VectorSubcoreMesh(core_axis_name="c", subcore_axis_name="s")
    sc_topk_scatter = functools.partial(
        pl.kernel,
        out_type=jax.ShapeDtypeStruct((_B * _B,), jnp.float32),
        mesh=mesh,
        scratch_types=[
            pltpu.VMEM((_BATCH * _B,), jnp.float32),
            pltpu.VMEM((_BATCH * _B,), jnp.float32),
            pltpu.VMEM((_RPW * _NCHUNK,), jnp.float32),
            pltpu.VMEM((_BATCH * _LANES,), jnp.int32),
        ],
        compiler_params=pltpu.CompilerParams(needs_layout_passes=False),
    )(_sc_topk_scatter_body)
    adjacency = pl.pallas_call(
        _adjacency_body,
        grid=(3, 3),
        in_specs=[pl.BlockSpec((_B, _B), lambda i, j: (0, 0))],
        out_specs=pl.BlockSpec((_B, _B), lambda i, j: (i, j)),
        out_shape=jax.ShapeDtypeStruct((3 * _B, 3 * _B), jnp.float32),
        scratch_shapes=[
            pltpu.VMEM((_B, _B), jnp.float32),
        ],
    )
    return dist, sc_topk_scatter, adjacency


@jax.jit
def kernel(nodes_t, nodes_a, nodes_v, batch_size):
    del batch_size
    t = nodes_t.astype(jnp.float32)
    a = nodes_a.astype(jnp.float32)
    v = nodes_v.astype(jnp.float32)
    dist_fn, sc_fn, adj_fn = _build_calls()
    dist, cm = dist_fn(t, a, v, t.T, a.T, v.T)
    ct_flat = sc_fn(dist.reshape(-1), cm.T.reshape(-1))
    adjacency = adj_fn(ct_flat.reshape(_B, _B))
    nodes_list = jnp.concatenate([t, a, v], axis=0)
    return adjacency, nodes_list


# speedup vs baseline: 1.6978x; 1.1077x over previous
import functools

import jax
import jax.numpy as jnp
from jax import lax
from jax.experimental import pallas as pl
from jax.experimental.pallas import tpu as pltpu
from jax.experimental.pallas import tpu_sc as plsc

_B = 1024
_EMB = 128
_K = 10
_RT = 256
_NW = 32
_RPW = _B // _NW
_BATCH = 8
_LANES = 16
_NCHUNK = _B // _LANES
_TOPC = 11


def _dist_body(t_ref, a_ref, v_ref, tt_ref, at_ref, vt_ref, dist_ref, cm_ref,
               h_scr, ht_scr):
    step = pl.program_id(0)

    C = jnp.float32(0.333984375)

    def _h(x, y, z):
        xb = x.astype(jnp.bfloat16).astype(jnp.float32)
        yb = y.astype(jnp.bfloat16).astype(jnp.float32)
        zb = z.astype(jnp.bfloat16).astype(jnp.float32)
        return (C * xb + C * yb) + C * zb

    @pl.when(step == 0)
    def _():
        h_scr[...] = _h(t_ref[...], a_ref[...], v_ref[...])
        ht_scr[...] = _h(tt_ref[...], at_ref[...], vt_ref[...])

    x = h_scr[pl.ds(step * _RT, _RT), :]

    UK = 16
    sub = lax.broadcasted_iota(jnp.int32, (UK, _EMB), 0)
    lane = lax.broadcasted_iota(jnp.int32, (UK, _EMB), 1)

    def kbody(kk, acc):
        yblk = ht_scr[pl.ds(kk * UK, UK), :]
        oh = jnp.where(lane == kk * UK + sub, 1.0, 0.0)
        xblk = lax.dot_general(x, oh, (((1,), (1,)), ((), ())),
                               precision=lax.Precision.HIGHEST,
                               preferred_element_type=jnp.float32)
        part = jnp.zeros((_RT, _B), jnp.float32)
        for u in range(UK):
            xk = lax.slice(xblk, (0, u), (_RT, u + 1))
            yk = lax.slice(yblk, (u, 0), (u + 1, _B))
            part = part + jnp.abs(xk - yk)
        return acc + part

    acc = lax.fori_loop(0, _EMB // UK, kbody,
                        jnp.zeros((_RT, _B), jnp.float32))
    dist_ref[...] = acc
    cm_ref[...] = jnp.max(acc.reshape(_RT // _LANES, _LANES, _B), axis=1)


def _sc_topk_scatter_body(dist_hbm, cm_hbm, ct_hbm, din, cout, cmv, ixb):
    core = lax.axis_index("c")
    sub = lax.axis_index("s")
    wid = sub * 2 + core
    base = wid * _RPW
    lanes = lax.iota(jnp.int32, _LANES)
    ones16 = jnp.ones((_LANES,), jnp.float32)
    zeros16 = jnp.zeros((_LANES,), jnp.float32)

    pltpu.sync_copy(cm_hbm.at[pl.ds(base * _NCHUNK, _RPW * _NCHUNK)], cmv)

    def zbody(c, _):
        cout[pl.ds(c * _LANES, _LANES)] = zeros16
        return 0
    lax.fori_loop(0, _BATCH * _B // _LANES, zbody, 0)

    def bmerge(a, ai, b, bi):
        rb = lax.rev(b, (0,))
        rbi = lax.rev(bi, (0,))
        mv = jnp.maximum(a, rb)
        mi = jnp.where(a >= rb, ai, rbi)
        nk, nv = plsc.sort_key_val(mv, mi, descending=True)
        return nk, nv

    for b in range(_RPW // _BATCH):
        row0 = base + b * _BATCH
        pltpu.sync_copy(dist_hbm.at[pl.ds(row0 * _B, _BATCH * _B)], din)

        def row_body(i, _):
            r = row0 + i
            ms, ids = [], []
            li = b * _BATCH + i
            for k in range(_NCHUNK // _LANES):
                cid = k * _LANES + lanes
                mk = cmv[pl.ds(li * _NCHUNK + k * _LANES, _LANES)]
                sk, sid = plsc.sort_key_val(mk, cid, descending=True)
                ms.append(sk)
                ids.append(sid)
            t0, t0i = bmerge(ms[0], ids[0], ms[1], ids[1])
            t1, t1i = bmerge(ms[2], ids[2], ms[3], ids[3])
            _sm, smi = bmerge(t0, t0i, t1, t1i)

            def merge_step(k, carry):
                cur, curi = carry
                ck = jnp.max(jnp.where(lanes == k, smi, -1))
                vidx = ck * _LANES + lanes
                vals = plsc.load_gather(din, [i * _B + vidx])
                sv, si = plsc.sort_key_val(vals, vidx, descending=True)
                return bmerge(cur, curi, sv, si)

            cur, curi = lax.fori_loop(
                0, _TOPC, merge_step,
                (jnp.full((_LANES,), -1.0, jnp.float32),
                 jnp.zeros((_LANES,), jnp.int32)))

            cols = jnp.where(lanes < _K, curi, r)
            idxv = i * _B + cols
            plsc.store_scatter(cout, [idxv], ones16, mask=lanes <= _K)
            ixb[pl.ds(i * _LANES, _LANES)] = idxv
            return 0

        lax.fori_loop(0, _BATCH, row_body, 0)
        pltpu.sync_copy(cout, ct_hbm.at[pl.ds(row0 * _B, _BATCH * _B)])

        def unscatter(i, _):
            iv = ixb[pl.ds(i * _LANES, _LANES)]
            plsc.store_scatter(cout, [iv], zeros16, mask=lanes <= _K)
            return 0
        lax.fori_loop(0, _BATCH, unscatter, 0)


def _adjacency_body(ct_ref, out_ref, a_scr):
    i = pl.program_id(0)
    j = pl.program_id(1)

    @pl.when((i == 0) & (j == 0))
    def _():
        ct = ct_ref[...]
        s = jnp.sum(ct, axis=1, keepdims=True)
        w = ct / s
        a0 = lax.dot_general(w, ct, (((0,), (0,)), ((), ())),
                             precision=lax.Precision.HIGHEST,
                             preferred_element_type=jnp.float32)
        ones = jnp.ones((_B, 1), jnp.float32)
        dcol = lax.dot_general(ct, ones, (((0,), (0,)), ((), ())),
                               precision=lax.Precision.HIGHEST)
        ii = lax.broadcasted_iota(jnp.int32, (_B, _B), 0)
        jj = lax.broadcasted_iota(jnp.int32, (_B, _B), 1)
        eye = jnp.where(ii == jj, 1.0, 0.0)
        a_scr[...] = (a0 + eye) / (3.0 * (1.0 + dcol))

    out_ref[...] = a_scr[...]


@functools.cache
def _build_calls():
    full_spec = lambda shape: pl.BlockSpec(shape, lambda s: (0, 0))
    dist = pl.pallas_call(
        _dist_body,
        grid=(_B // _RT,),
        in_specs=[
            full_spec((_B, _EMB)), full_spec((_B, _EMB)), full_spec((_B, _EMB)),
            full_spec((_EMB, _B)), full_spec((_EMB, _B)), full_spec((_EMB, _B)),
        ],
        out_specs=[
            pl.BlockSpec((_RT, _B), lambda s: (s, 0)),
            pl.BlockSpec((_RT // _LANES, _B), lambda s: (s, 0)),
        ],
        out_shape=[
            jax.ShapeDtypeStruct((_B, _B), jnp.float32),
            jax.ShapeDtypeStruct((_NCHUNK, _B), jnp.float32),
        ],
        scratch_shapes=[
            pltpu.VMEM((_B, _EMB), jnp.float32),
            pltpu.VMEM((_EMB, _B), jnp.float32),
        ],
    )
    mesh = plsc.VectorSubcoreMesh(core_axis_name="c", subcore_axis_name="s")
    sc_topk_scatter = functools.partial(
        pl.kernel,
        out_type=jax.ShapeDtypeStruct((_B * _B,), jnp.float32),
        mesh=mesh,
        scratch_types=[
            pltpu.VMEM((_BATCH * _B,), jnp.float32),
            pltpu.VMEM((_BATCH * _B,), jnp.float32),
            pltpu.VMEM((_RPW * _NCHUNK,), jnp.float32),
            pltpu.VMEM((_BATCH * _LANES,), jnp.int32),
        ],
        compiler_params=pltpu.CompilerParams(needs_layout_passes=False),
    )(_sc_topk_scatter_body)
    adjacency = pl.pallas_call(
        _adjacency_body,
        grid=(3, 3),
        in_specs=[pl.BlockSpec((_B, _B), lambda i, j: (0, 0))],
        out_specs=pl.BlockSpec((_B, _B), lambda i, j: (i, j)),
        out_shape=jax.ShapeDtypeStruct((3 * _B, 3 * _B), jnp.float32),
        scratch_shapes=[
            pltpu.VMEM((_B, _B), jnp.float32),
        ],
    )
    return dist, sc_topk_scatter, adjacency


@jax.jit
def kernel(nodes_t, nodes_a, nodes_v, batch_size):
    del batch_size
    t = nodes_t.astype(jnp.float32)
    a = nodes_a.astype(jnp.float32)
    v = nodes_v.astype(jnp.float32)
    dist_fn, sc_fn, adj_fn = _build_calls()
    dist, cm = dist_fn(t, a, v, t.T, a.T, v.T)
    ct_flat = sc_fn(dist.reshape(-1), cm.T.reshape(-1))
    adjacency = adj_fn(ct_flat.reshape(_B, _B))
    nodes_list = jnp.concatenate([t, a, v], axis=0)
    return adjacency, nodes_list


# speedup vs baseline: 1.7445x; 1.0275x over previous
import functools

import jax
import jax.numpy as jnp
from jax import lax
from jax.experimental import pallas as pl
from jax.experimental.pallas import tpu as pltpu
from jax.experimental.pallas import tpu_sc as plsc

_B = 1024
_EMB = 128
_K = 10
_RT = 256
_NW = 32
_RPW = _B // _NW
_BATCH = 8
_LANES = 16
_NCHUNK = _B // _LANES
_TOPC = 11


def _dist_body(t_ref, a_ref, v_ref, tt_ref, at_ref, vt_ref, dist_ref, cm_ref,
               h_scr, ht_scr):
    step = pl.program_id(0)

    C = jnp.float32(0.333984375)

    def _h(x, y, z):
        xb = x.astype(jnp.bfloat16).astype(jnp.float32)
        yb = y.astype(jnp.bfloat16).astype(jnp.float32)
        zb = z.astype(jnp.bfloat16).astype(jnp.float32)
        return (C * xb + C * yb) + C * zb

    @pl.when(step == 0)
    def _():
        h_scr[...] = _h(t_ref[...], a_ref[...], v_ref[...])
        ht_scr[...] = _h(tt_ref[...], at_ref[...], vt_ref[...])

    x = h_scr[pl.ds(step * _RT, _RT), :]

    UK = 32
    sub = lax.broadcasted_iota(jnp.int32, (UK, _EMB), 0)
    lane = lax.broadcasted_iota(jnp.int32, (UK, _EMB), 1)

    def kbody(kk, acc):
        yblk = ht_scr[pl.ds(kk * UK, UK), :]
        oh = jnp.where(lane == kk * UK + sub, 1.0, 0.0)
        xblk = lax.dot_general(x, oh, (((1,), (1,)), ((), ())),
                               precision=lax.Precision.HIGHEST,
                               preferred_element_type=jnp.float32)
        part = jnp.zeros((_RT, _B), jnp.float32)
        for u in range(UK):
            xk = lax.slice(xblk, (0, u), (_RT, u + 1))
            yk = lax.slice(yblk, (u, 0), (u + 1, _B))
            part = part + jnp.abs(xk - yk)
        return acc + part

    acc = lax.fori_loop(0, _EMB // UK, kbody,
                        jnp.zeros((_RT, _B), jnp.float32))
    dist_ref[...] = acc
    cm_ref[...] = jnp.max(acc.reshape(_RT // _LANES, _LANES, _B), axis=1)


def _sc_topk_scatter_body(dist_hbm, cm_hbm, ct_hbm, din, cout, cmv, ixb):
    core = lax.axis_index("c")
    sub = lax.axis_index("s")
    wid = sub * 2 + core
    base = wid * _RPW
    lanes = lax.iota(jnp.int32, _LANES)
    ones16 = jnp.ones((_LANES,), jnp.float32)
    zeros16 = jnp.zeros((_LANES,), jnp.float32)

    pltpu.sync_copy(cm_hbm.at[pl.ds(base * _NCHUNK, _RPW * _NCHUNK)], cmv)

    def zbody(c, _):
        cout[pl.ds(c * _LANES, _LANES)] = zeros16
        return 0
    lax.fori_loop(0, _BATCH * _B // _LANES, zbody, 0)

    def bmerge(a, ai, b, bi):
        rb = lax.rev(b, (0,))
        rbi = lax.rev(bi, (0,))
        mv = jnp.maximum(a, rb)
        mi = jnp.where(a >= rb, ai, rbi)
        nk, nv = plsc.sort_key_val(mv, mi, descending=True)
        return nk, nv

    for b in range(_RPW // _BATCH):
        row0 = base + b * _BATCH
        pltpu.sync_copy(dist_hbm.at[pl.ds(row0 * _B, _BATCH * _B)], din)

        def row_body(i, _):
            r = row0 + i
            ms, ids = [], []
            li = b * _BATCH + i
            for k in range(_NCHUNK // _LANES):
                cid = k * _LANES + lanes
                mk = cmv[pl.ds(li * _NCHUNK + k * _LANES, _LANES)]
                sk, sid = plsc.sort_key_val(mk, cid, descending=True)
                ms.append(sk)
                ids.append(sid)
            t0, t0i = bmerge(ms[0], ids[0], ms[1], ids[1])
            t1, t1i = bmerge(ms[2], ids[2], ms[3], ids[3])
            _sm, smi = bmerge(t0, t0i, t1, t1i)

            def merge_step(k, carry):
                cur, curi = carry
                ck = jnp.max(jnp.where(lanes == k, smi, -1))
                vidx = ck * _LANES + lanes
                vals = plsc.load_gather(din, [i * _B + vidx])
                sv, si = plsc.sort_key_val(vals, vidx, descending=True)
                return bmerge(cur, curi, sv, si)

            cur, curi = lax.fori_loop(
                0, _TOPC, merge_step,
                (jnp.full((_LANES,), -1.0, jnp.float32),
                 jnp.zeros((_LANES,), jnp.int32)))

            cols = jnp.where(lanes < _K, curi, r)
            idxv = i * _B + cols
            plsc.store_scatter(cout, [idxv], ones16, mask=lanes <= _K)
            ixb[pl.ds(i * _LANES, _LANES)] = idxv
            return 0

        lax.fori_loop(0, _BATCH, row_body, 0)
        pltpu.sync_copy(cout, ct_hbm.at[pl.ds(row0 * _B, _BATCH * _B)])

        def unscatter(i, _):
            iv = ixb[pl.ds(i * _LANES, _LANES)]
            plsc.store_scatter(cout, [iv], zeros16, mask=lanes <= _K)
            return 0
        lax.fori_loop(0, _BATCH, unscatter, 0)


def _adjacency_body(ct_ref, out_ref, a_scr):
    i = pl.program_id(0)
    j = pl.program_id(1)

    @pl.when((i == 0) & (j == 0))
    def _():
        ct = ct_ref[...]
        s = jnp.sum(ct, axis=1, keepdims=True)
        w = ct / s
        a0 = lax.dot_general(w, ct, (((0,), (0,)), ((), ())),
                             precision=lax.Precision.HIGHEST,
                             preferred_element_type=jnp.float32)
        ones = jnp.ones((_B, 1), jnp.float32)
        dcol = lax.dot_general(ct, ones, (((0,), (0,)), ((), ())),
                               precision=lax.Precision.HIGHEST)
        ii = lax.broadcasted_iota(jnp.int32, (_B, _B), 0)
        jj = lax.broadcasted_iota(jnp.int32, (_B, _B), 1)
        eye = jnp.where(ii == jj, 1.0, 0.0)
        a_scr[...] = (a0 + eye) / (3.0 * (1.0 + dcol))

    out_ref[...] = a_scr[...]


@functools.cache
def _build_calls():
    full_spec = lambda shape: pl.BlockSpec(shape, lambda s: (0, 0))
    dist = pl.pallas_call(
        _dist_body,
        grid=(_B // _RT,),
        in_specs=[
            full_spec((_B, _EMB)), full_spec((_B, _EMB)), full_spec((_B, _EMB)),
            full_spec((_EMB, _B)), full_spec((_EMB, _B)), full_spec((_EMB, _B)),
        ],
        out_specs=[
            pl.BlockSpec((_RT, _B), lambda s: (s, 0)),
            pl.BlockSpec((_RT // _LANES, _B), lambda s: (s, 0)),
        ],
        out_shape=[
            jax.ShapeDtypeStruct((_B, _B), jnp.float32),
            jax.ShapeDtypeStruct((_NCHUNK, _B), jnp.float32),
        ],
        scratch_shapes=[
            pltpu.VMEM((_B, _EMB), jnp.float32),
            pltpu.VMEM((_EMB, _B), jnp.float32),
        ],
    )
    mesh = plsc.VectorSubcoreMesh(core_axis_name="c", subcore_axis_name="s")
    sc_topk_scatter = functools.partial(
        pl.kernel,
        out_type=jax.ShapeDtypeStruct((_B * _B,), jnp.float32),
        mesh=mesh,
        scratch_types=[
            pltpu.VMEM((_BATCH * _B,), jnp.float32),
            pltpu.VMEM((_BATCH * _B,), jnp.float32),
            pltpu.VMEM((_RPW * _NCHUNK,), jnp.float32),
            pltpu.VMEM((_BATCH * _LANES,), jnp.int32),
        ],
        compiler_params=pltpu.CompilerParams(needs_layout_passes=False),
    )(_sc_topk_scatter_body)
    adjacency = pl.pallas_call(
        _adjacency_body,
        grid=(3, 3),
        in_specs=[pl.BlockSpec((_B, _B), lambda i, j: (0, 0))],
        out_specs=pl.BlockSpec((_B, _B), lambda i, j: (i, j)),
        out_shape=jax.ShapeDtypeStruct((3 * _B, 3 * _B), jnp.float32),
        scratch_shapes=[
            pltpu.VMEM((_B, _B), jnp.float32),
        ],
    )
    return dist, sc_topk_scatter, adjacency


@jax.jit
def kernel(nodes_t, nodes_a, nodes_v, batch_size):
    del batch_size
    t = nodes_t.astype(jnp.float32)
    a = nodes_a.astype(jnp.float32)
    v = nodes_v.astype(jnp.float32)
    dist_fn, sc_fn, adj_fn = _build_calls()
    dist, cm = dist_fn(t, a, v, t.T, a.T, v.T)
    ct_flat = sc_fn(dist.reshape(-1), cm.T.reshape(-1))
    adjacency = adj_fn(ct_flat.reshape(_B, _B))
    nodes_list = jnp.concatenate([t, a, v], axis=0)
    return adjacency, nodes_list


# speedup vs baseline: 1.7650x; 1.0117x over previous
import functools

import jax
import jax.numpy as jnp
from jax import lax
from jax.experimental import pallas as pl
from jax.experimental.pallas import tpu as pltpu
from jax.experimental.pallas import tpu_sc as plsc

_B = 1024
_EMB = 128
_K = 10
_RT = 256
_NW = 32
_RPW = _B // _NW
_BATCH = 8
_LANES = 16
_NCHUNK = _B // _LANES
_TOPC = 11


def _dist_body(t_ref, a_ref, v_ref, tt_ref, at_ref, vt_ref, dist_ref, cm_ref,
               h_scr, ht_scr):
    step = pl.program_id(0)

    C = jnp.float32(0.333984375)

    def _h(x, y, z):
        xb = x.astype(jnp.bfloat16).astype(jnp.float32)
        yb = y.astype(jnp.bfloat16).astype(jnp.float32)
        zb = z.astype(jnp.bfloat16).astype(jnp.float32)
        return (C * xb + C * yb) + C * zb

    @pl.when(step == 0)
    def _():
        h_scr[...] = _h(t_ref[...], a_ref[...], v_ref[...])
        ht_scr[...] = _h(tt_ref[...], at_ref[...], vt_ref[...])

    x = h_scr[pl.ds(step * _RT, _RT), :]

    UK = 64
    sub = lax.broadcasted_iota(jnp.int32, (UK, _EMB), 0)
    lane = lax.broadcasted_iota(jnp.int32, (UK, _EMB), 1)

    def kbody(kk, acc):
        yblk = ht_scr[pl.ds(kk * UK, UK), :]
        oh = jnp.where(lane == kk * UK + sub, 1.0, 0.0)
        xblk = lax.dot_general(x, oh, (((1,), (1,)), ((), ())),
                               precision=lax.Precision.HIGHEST,
                               preferred_element_type=jnp.float32)
        part = jnp.zeros((_RT, _B), jnp.float32)
        for u in range(UK):
            xk = lax.slice(xblk, (0, u), (_RT, u + 1))
            yk = lax.slice(yblk, (u, 0), (u + 1, _B))
            part = part + jnp.abs(xk - yk)
        return acc + part

    acc = lax.fori_loop(0, _EMB // UK, kbody,
                        jnp.zeros((_RT, _B), jnp.float32))
    dist_ref[...] = acc
    cm_ref[...] = jnp.max(acc.reshape(_RT // _LANES, _LANES, _B), axis=1)


def _sc_topk_scatter_body(dist_hbm, cm_hbm, ct_hbm, din, cout, cmv, ixb):
    core = lax.axis_index("c")
    sub = lax.axis_index("s")
    wid = sub * 2 + core
    base = wid * _RPW
    lanes = lax.iota(jnp.int32, _LANES)
    ones16 = jnp.ones((_LANES,), jnp.float32)
    zeros16 = jnp.zeros((_LANES,), jnp.float32)

    pltpu.sync_copy(cm_hbm.at[pl.ds(base * _NCHUNK, _RPW * _NCHUNK)], cmv)

    def zbody(c, _):
        cout[pl.ds(c * _LANES, _LANES)] = zeros16
        return 0
    lax.fori_loop(0, _BATCH * _B // _LANES, zbody, 0)

    def bmerge(a, ai, b, bi):
        rb = lax.rev(b, (0,))
        rbi = lax.rev(bi, (0,))
        mv = jnp.maximum(a, rb)
        mi = jnp.where(a >= rb, ai, rbi)
        nk, nv = plsc.sort_key_val(mv, mi, descending=True)
        return nk, nv

    for b in range(_RPW // _BATCH):
        row0 = base + b * _BATCH
        pltpu.sync_copy(dist_hbm.at[pl.ds(row0 * _B, _BATCH * _B)], din)

        def row_body(i, _):
            r = row0 + i
            ms, ids = [], []
            li = b * _BATCH + i
            for k in range(_NCHUNK // _LANES):
                cid = k * _LANES + lanes
                mk = cmv[pl.ds(li * _NCHUNK + k * _LANES, _LANES)]
                sk, sid = plsc.sort_key_val(mk, cid, descending=True)
                ms.append(sk)
                ids.append(sid)
            t0, t0i = bmerge(ms[0], ids[0], ms[1], ids[1])
            t1, t1i = bmerge(ms[2], ids[2], ms[3], ids[3])
            _sm, smi = bmerge(t0, t0i, t1, t1i)

            def merge_step(k, carry):
                cur, curi = carry
                ck = jnp.max(jnp.where(lanes == k, smi, -1))
                vidx = ck * _LANES + lanes
                vals = plsc.load_gather(din, [i * _B + vidx])
                sv, si = plsc.sort_key_val(vals, vidx, descending=True)
                return bmerge(cur, curi, sv, si)

            cur, curi = lax.fori_loop(
                0, _TOPC, merge_step,
                (jnp.full((_LANES,), -1.0, jnp.float32),
                 jnp.zeros((_LANES,), jnp.int32)))

            cols = jnp.where(lanes < _K, curi, r)
            idxv = i * _B + cols
            plsc.store_scatter(cout, [idxv], ones16, mask=lanes <= _K)
            ixb[pl.ds(i * _LANES, _LANES)] = idxv
            return 0

        lax.fori_loop(0, _BATCH, row_body, 0)
        pltpu.sync_copy(cout, ct_hbm.at[pl.ds(row0 * _B, _BATCH * _B)])

        def unscatter(i, _):
            iv = ixb[pl.ds(i * _LANES, _LANES)]
            plsc.store_scatter(cout, [iv], zeros16, mask=lanes <= _K)
            return 0
        lax.fori_loop(0, _BATCH, unscatter, 0)


def _adjacency_body(ct_ref, out_ref, a_scr):
    i = pl.program_id(0)
    j = pl.program_id(1)

    @pl.when((i == 0) & (j == 0))
    def _():
        ct = ct_ref[...]
        s = jnp.sum(ct, axis=1, keepdims=True)
        w = ct / s
        a0 = lax.dot_general(w, ct, (((0,), (0,)), ((), ())),
                             precision=lax.Precision.HIGHEST,
                             preferred_element_type=jnp.float32)
        ones = jnp.ones((_B, 1), jnp.float32)
        dcol = lax.dot_general(ct, ones, (((0,), (0,)), ((), ())),
                               precision=lax.Precision.HIGHEST)
        ii = lax.broadcasted_iota(jnp.int32, (_B, _B), 0)
        jj = lax.broadcasted_iota(jnp.int32, (_B, _B), 1)
        eye = jnp.where(ii == jj, 1.0, 0.0)
        a_scr[...] = (a0 + eye) / (3.0 * (1.0 + dcol))

    out_ref[...] = a_scr[...]


@functools.cache
def _build_calls():
    full_spec = lambda shape: pl.BlockSpec(shape, lambda s: (0, 0))
    dist = pl.pallas_call(
        _dist_body,
        grid=(_B // _RT,),
        in_specs=[
            full_spec((_B, _EMB)), full_spec((_B, _EMB)), full_spec((_B, _EMB)),
            full_spec((_EMB, _B)), full_spec((_EMB, _B)), full_spec((_EMB, _B)),
        ],
        out_specs=[
            pl.BlockSpec((_RT, _B), lambda s: (s, 0)),
            pl.BlockSpec((_RT // _LANES, _B), lambda s: (s, 0)),
        ],
        out_shape=[
            jax.ShapeDtypeStruct((_B, _B), jnp.float32),
            jax.ShapeDtypeStruct((_NCHUNK, _B), jnp.float32),
        ],
        scratch_shapes=[
            pltpu.VMEM((_B, _EMB), jnp.float32),
            pltpu.VMEM((_EMB, _B), jnp.float32),
        ],
    )
    mesh = plsc.VectorSubcoreMesh(core_axis_name="c", subcore_axis_name="s")
    sc_topk_scatter = functools.partial(
        pl.kernel,
        out_type=jax.ShapeDtypeStruct((_B * _B,), jnp.float32),
        mesh=mesh,
        scratch_types=[
            pltpu.VMEM((_BATCH * _B,), jnp.float32),
            pltpu.VMEM((_BATCH * _B,), jnp.float32),
            pltpu.VMEM((_RPW * _NCHUNK,), jnp.float32),
            pltpu.VMEM((_BATCH * _LANES,), jnp.int32),
        ],
        compiler_params=pltpu.CompilerParams(needs_layout_passes=False),
    )(_sc_topk_scatter_body)
    adjacency = pl.pallas_call(
        _adjacency_body,
        grid=(3, 3),
        in_specs=[pl.BlockSpec((_B, _B), lambda i, j: (0, 0))],
        out_specs=pl.BlockSpec((_B, _B), lambda i, j: (i, j)),
        out_shape=jax.ShapeDtypeStruct((3 * _B, 3 * _B), jnp.float32),
        scratch_shapes=[
            pltpu.VMEM((_B, _B), jnp.float32),
        ],
    )
    return dist, sc_topk_scatter, adjacency


@jax.jit
def kernel(nodes_t, nodes_a, nodes_v, batch_size):
    del batch_size
    t = nodes_t.astype(jnp.float32)
    a = nodes_a.astype(jnp.float32)
    v = nodes_v.astype(jnp.float32)
    dist_fn, sc_fn, adj_fn = _build_calls()
    dist, cm = dist_fn(t, a, v, t.T, a.T, v.T)
    ct_flat = sc_fn(dist.reshape(-1), cm.T.reshape(-1))
    adjacency = adj_fn(ct_flat.reshape(_B, _B))
    nodes_list = jnp.concatenate([t, a, v], axis=0)
    return adjacency, nodes_list
